# baseline (speedup 1.0000x reference)
import functools

import jax
import jax.numpy as jnp
from jax import lax
from jax.experimental import pallas as pl
from jax.experimental.pallas import tpu as pltpu
from jax.experimental.pallas import tpu_sc as plsc

_L = 16
_CHUNK = 128


@functools.lru_cache(maxsize=None)
def _sc_gather_prod(total, rank, table_rows):
    info = plsc.get_sparse_core_info()
    num_workers = info.num_cores * info.num_subcores
    per_w = total // num_workers
    assert per_w % _CHUNK == 0
    n_chunks = per_w // _CHUNK

    mesh = plsc.VectorSubcoreMesh(core_axis_name="c", subcore_axis_name="s")

    buf = lambda shape, dt: pltpu.VMEM(shape, dt)

    @functools.partial(
        pl.kernel,
        mesh=mesh,
        compiler_params=pltpu.CompilerParams(use_tc_tiling_on_sc=False),
        out_type=jax.ShapeDtypeStruct((total, _L), jnp.float32),
        scratch_types=[
            buf((per_w,), jnp.float32),
            buf((per_w,), jnp.float32),
            buf((per_w,), jnp.float32),
            buf((_CHUNK,), jnp.int32),
            buf((_CHUNK,), jnp.int32),
            buf((_CHUNK,), jnp.int32),
            buf((_CHUNK, rank), jnp.float32),
            buf((_CHUNK, rank), jnp.float32),
            buf((_CHUNK, rank), jnp.float32),
            buf((_CHUNK, _L), jnp.float32),
            pltpu.SemaphoreType.DMA,
        ],
    )
    def sc_fn(xf, yf, tf, At, Bt, Ct, p_out,
              fx, fy, ft, ix, iy, it, rA, rB, rC, pbuf, semg):
        wid = lax.axis_index("s") * info.num_cores + lax.axis_index("c")
        base = wid * per_w
        pltpu.sync_copy(xf.at[pl.ds(base, per_w)], fx)
        pltpu.sync_copy(yf.at[pl.ds(base, per_w)], fy)
        pltpu.sync_copy(tf.at[pl.ds(base, per_w)], ft)

        xscale = jnp.float32(table_rows - 1)
        yscale = jnp.float32(table_rows)
        hi = table_rows - 1

        lo = pl.ds(0, _L)
        hi_sl = pl.ds(_L, _L)

        def chunk_body(c, carry):
            coff = c * _CHUNK

            @plsc.parallel_loop(0, _CHUNK // _L, unroll=4)
            def _(gi):
                src = pl.ds(coff + gi * _L, _L)
                dst = pl.ds(gi * _L, _L)
                ix[dst] = jnp.clip((fx[src] * xscale).astype(jnp.int32), 0, hi)
                iy[dst] = jnp.clip((fy[src] * yscale - 1.0).astype(jnp.int32), 0, hi)
                it[dst] = jnp.clip((ft[src] * yscale - 1.0).astype(jnp.int32), 0, hi)

            pltpu.async_copy(At.at[ix], rA, semg)
            pltpu.async_copy(Bt.at[iy], rB, semg)
            pltpu.async_copy(Ct.at[it], rC, semg)
            pltpu.make_async_copy(At.at[ix], rA, semg).wait()
            pltpu.make_async_copy(Bt.at[iy], rB, semg).wait()
            pltpu.make_async_copy(Ct.at[it], rC, semg).wait()

            @plsc.parallel_loop(0, _CHUNK, unroll=8)
            def _(j):
                p = (rA[j, lo] * rB[j, lo] * rC[j, lo]
                     + rA[j, hi_sl] * rB[j, hi_sl] * rC[j, hi_sl])
                pbuf[j, lo] = p

            pltpu.sync_copy(pbuf, p_out.at[pl.ds(base + coff, _CHUNK)])
            return carry

        lax.fori_loop(0, n_chunks, chunk_body, 0)

    return sc_fn


@functools.lru_cache(maxsize=None)
def _tc_linear(n, k, feat):
    blk = 1024

    def mm(p_ref, w_ref, b_ref, o_ref):
        o_ref[...] = (
            jnp.dot(p_ref[...], w_ref[...], preferred_element_type=jnp.float32)
            + b_ref[...]
        )

    return pl.pallas_call(
        mm,
        grid=(n // blk,),
        in_specs=[
            pl.BlockSpec((blk, k), lambda i: (i, 0)),
            pl.BlockSpec((k, feat), lambda i: (0, 0)),
            pl.BlockSpec((1, feat), lambda i: (0, 0)),
        ],
        out_specs=pl.BlockSpec((blk, feat), lambda i: (i, 0)),
        out_shape=jax.ShapeDtypeStruct((n, feat), jnp.float32),
    )


def kernel(x_idx, y_idx, t_idx, A, B, C, W, b):
    rank, n = x_idx.shape
    table_rows = A.shape[1]
    feat = W.shape[0]
    total = rank * n

    xf = x_idx.T.reshape(total)
    yf = y_idx.T.reshape(total)
    tf = t_idx.T.reshape(total)
    At = A.T
    Bt = B.T
    Ct = C.T

    p = _sc_gather_prod(total, rank, table_rows)(xf, yf, tf, At, Bt, Ct)

    w2 = jnp.broadcast_to(W.T[:, None, :], (rank, _L, feat)).reshape(rank * _L, feat)
    return _tc_linear(n, rank * _L, feat)(p.reshape(n, rank * _L), w2, b.reshape(1, feat))


# speedup vs baseline: 1.5568x; 1.5568x over previous
import functools

import jax
import jax.numpy as jnp
from jax import lax
from jax.experimental import pallas as pl
from jax.experimental.pallas import tpu as pltpu
from jax.experimental.pallas import tpu_sc as plsc

_L = 16
_CHUNK = 128
_SETS = 4


@functools.lru_cache(maxsize=None)
def _sc_gather_prod(total, rank, table_rows):
    info = plsc.get_sparse_core_info()
    num_workers = info.num_cores * info.num_subcores
    per_w = total // num_workers
    assert per_w % (_SETS * _CHUNK) == 0
    n_chunks = per_w // _CHUNK
    rounds = n_chunks // _SETS

    mesh = plsc.VectorSubcoreMesh(core_axis_name="c", subcore_axis_name="s")

    buf = lambda shape, dt: pltpu.VMEM(shape, dt)

    @functools.partial(
        pl.kernel,
        mesh=mesh,
        compiler_params=pltpu.CompilerParams(use_tc_tiling_on_sc=False),
        out_type=jax.ShapeDtypeStruct((total, _L), jnp.float32),
        scratch_types=[
            buf((per_w,), jnp.float32),
            buf((per_w,), jnp.float32),
            buf((per_w,), jnp.float32),
            [buf((_CHUNK,), jnp.int32) for _ in range(_SETS)],
            [buf((_CHUNK,), jnp.int32) for _ in range(_SETS)],
            [buf((_CHUNK,), jnp.int32) for _ in range(_SETS)],
            [buf((_CHUNK, rank), jnp.float32) for _ in range(_SETS)],
            [buf((_CHUNK, rank), jnp.float32) for _ in range(_SETS)],
            [buf((_CHUNK, rank), jnp.float32) for _ in range(_SETS)],
            [buf((_CHUNK, _L), jnp.float32) for _ in range(_SETS)],
            [pltpu.SemaphoreType.DMA for _ in range(_SETS)],
            [pltpu.SemaphoreType.DMA for _ in range(_SETS)],
        ],
    )
    def sc_fn(xf, yf, tf, At, Bt, Ct, p_out,
              fx, fy, ft, ix, iy, it, rA, rB, rC, pbuf, semg, semw):
        wid = lax.axis_index("s") * info.num_cores + lax.axis_index("c")
        base = wid * per_w
        pltpu.sync_copy(xf.at[pl.ds(base, per_w)], fx)
        pltpu.sync_copy(yf.at[pl.ds(base, per_w)], fy)
        pltpu.sync_copy(tf.at[pl.ds(base, per_w)], ft)

        xscale = jnp.float32(table_rows - 1)
        yscale = jnp.float32(table_rows)
        hi = table_rows - 1

        def quantize(c, b):
            coff = c * _CHUNK

            @plsc.parallel_loop(0, _CHUNK // _L, unroll=4)
            def _(gi):
                src = pl.ds(coff + gi * _L, _L)
                dst = pl.ds(gi * _L, _L)
                ix[b][dst] = jnp.clip((fx[src] * xscale).astype(jnp.int32), 0, hi)
                iy[b][dst] = jnp.clip((fy[src] * yscale - 1.0).astype(jnp.int32), 0, hi)
                it[b][dst] = jnp.clip((ft[src] * yscale - 1.0).astype(jnp.int32), 0, hi)

        def fire(b):
            pltpu.async_copy(At.at[ix[b]], rA[b], semg[b])
            pltpu.async_copy(Bt.at[iy[b]], rB[b], semg[b])
            pltpu.async_copy(Ct.at[it[b]], rC[b], semg[b])

        def waitg(b):
            pltpu.make_async_copy(At.at[ix[b]], rA[b], semg[b]).wait()
            pltpu.make_async_copy(Bt.at[iy[b]], rB[b], semg[b]).wait()
            pltpu.make_async_copy(Ct.at[it[b]], rC[b], semg[b]).wait()

        lo = pl.ds(0, _L)
        hi_sl = pl.ds(_L, _L)

        def compute(b):
            @plsc.parallel_loop(0, _CHUNK, unroll=8)
            def _(j):
                p = (rA[b][j, lo] * rB[b][j, lo] * rC[b][j, lo]
                     + rA[b][j, hi_sl] * rB[b][j, hi_sl] * rC[b][j, hi_sl])
                pbuf[b][j, lo] = p

        def fire_out(c, b):
            pltpu.async_copy(pbuf[b], p_out.at[pl.ds(base + c * _CHUNK, _CHUNK)], semw[b])

        def wait_out(c, b):
            pltpu.make_async_copy(
                pbuf[b], p_out.at[pl.ds(base + c * _CHUNK, _CHUNK)], semw[b]
            ).wait()

        for s in range(_SETS):
            quantize(s, s)
            fire(s)

        def round_body(r, carry):
            for s in range(_SETS):
                c = r * _SETS + s
                waitg(s)

                @pl.when(r >= 2)
                def _():
                    wait_out(c - 2 * _SETS, s)

                compute(s)
                fire_out(c - _SETS, s)
                quantize(c, s)
                fire(s)
            return carry

        lax.fori_loop(1, rounds, round_body, 0)

        for s in range(_SETS):
            c = n_chunks - _SETS + s
            waitg(s)
            if rounds >= 2:
                wait_out(c - _SETS, s)
            compute(s)
            fire_out(c, s)
        for s in range(_SETS):
            wait_out(n_chunks - _SETS + s, s)

    return sc_fn


@functools.lru_cache(maxsize=None)
def _tc_linear(n, k, feat):
    blk = 1024

    def mm(p_ref, w_ref, b_ref, o_ref):
        o_ref[...] = (
            jnp.dot(p_ref[...], w_ref[...], preferred_element_type=jnp.float32)
            + b_ref[...]
        )

    return pl.pallas_call(
        mm,
        grid=(n // blk,),
        in_specs=[
            pl.BlockSpec((blk, k), lambda i: (i, 0)),
            pl.BlockSpec((k, feat), lambda i: (0, 0)),
            pl.BlockSpec((1, feat), lambda i: (0, 0)),
        ],
        out_specs=pl.BlockSpec((blk, feat), lambda i: (i, 0)),
        out_shape=jax.ShapeDtypeStruct((n, feat), jnp.float32),
    )


def kernel(x_idx, y_idx, t_idx, A, B, C, W, b):
    rank, n = x_idx.shape
    table_rows = A.shape[1]
    feat = W.shape[0]
    total = rank * n

    xf = x_idx.T.reshape(total)
    yf = y_idx.T.reshape(total)
    tf = t_idx.T.reshape(total)
    At = A.T
    Bt = B.T
    Ct = C.T

    p = _sc_gather_prod(total, rank, table_rows)(xf, yf, tf, At, Bt, Ct)

    w2 = jnp.broadcast_to(W.T[:, None, :], (rank, _L, feat)).reshape(rank * _L, feat)
    return _tc_linear(n, rank * _L, feat)(p.reshape(n, rank * _L), w2, b.reshape(1, feat))
